# baseline (speedup 1.0000x reference)
import functools

import jax
import jax.numpy as jnp
from jax import lax
from jax.experimental import pallas as pl
from jax.experimental.pallas import tpu as pltpu
from jax.experimental.pallas import tpu_sc as plsc

_NUM_CORES = 2
_NUM_SUBCORES = 16
_NUM_WORKERS = _NUM_CORES * _NUM_SUBCORES
_LANES = 16
_NSLOT = 8


def kernel(x, table):
    (batch,) = x.shape
    n_rows, embed_dim = table.shape
    table_t = table.T
    b_per_w = batch // _NUM_WORKERS
    mesh = plsc.VectorSubcoreMesh(core_axis_name="c", subcore_axis_name="s")

    @functools.partial(
        pl.kernel,
        mesh=mesh,
        out_type=jax.ShapeDtypeStruct((embed_dim, batch), table.dtype),
        scratch_types=[
            pltpu.VMEM((b_per_w + _LANES,), jnp.int32),
            pltpu.VMEM((_NSLOT, embed_dim, 128), jnp.float32),
            pltpu.VMEM((b_per_w, embed_dim), jnp.float32),
            pltpu.VMEM((embed_dim, b_per_w), jnp.float32),
            pltpu.SemaphoreType.DMA((_NSLOT,)),
        ],
        compiler_params=pltpu.CompilerParams(needs_layout_passes=False),
    )
    def emb(x_hbm, table_hbm, out_hbm, idx_v, stage_v, res_v, outb_v, sems):
        wid = lax.axis_index("s") * _NUM_CORES + lax.axis_index("c")
        base = wid * b_per_w
        pltpu.sync_copy(x_hbm.at[pl.ds(base, b_per_w)], idx_v.at[pl.ds(0, b_per_w)])
        idx_v[pl.ds(b_per_w, _LANES)] = jnp.zeros((_LANES,), jnp.int32)
        cvec = lax.iota(jnp.int32, _LANES)

        def issue(k_lane, slot, ivec16):
            col = pl.multiple_of((ivec16[k_lane] // 128) * 128, 128)
            pltpu.async_copy(
                table_hbm.at[:, pl.ds(col, 128)],
                stage_v.at[slot],
                sems.at[slot],
            )

        ivec0 = idx_v[pl.ds(0, _LANES)]
        for k in range(_NSLOT):
            issue(k, k, ivec0)

        n_hg = b_per_w // _NSLOT

        def group_body(hg, _):
            ivec = idx_v[pl.ds(hg * _NSLOT, _LANES)]
            nb = lax.rem((hg + 1) * _NSLOT, b_per_w)
            ivec_next = idx_v[pl.ds(nb, _LANES)]
            lanes = ivec & 127

            for k in range(_NSLOT):
                pltpu.make_async_copy(
                    table_hbm.at[:, pl.ds(0, 128)],
                    stage_v.at[k],
                    sems.at[k],
                ).wait()
                lane = lanes[k]
                va = plsc.load_gather(
                    stage_v, [jnp.full((_LANES,), k, jnp.int32), cvec,
                              jnp.full((_LANES,), lane, jnp.int32)]
                )
                vb = plsc.load_gather(
                    stage_v, [jnp.full((_LANES,), k, jnp.int32), cvec + _LANES,
                              jnp.full((_LANES,), lane, jnp.int32)]
                )
                b_loc = hg * _NSLOT + k
                res_v[b_loc, pl.ds(0, _LANES)] = va
                res_v[b_loc, pl.ds(_LANES, _LANES)] = vb
                issue(k, k, ivec_next)

            return ()

        lax.fori_loop(0, n_hg, group_body, (), unroll=1)

        for k in range(_NSLOT):
            pltpu.make_async_copy(
                table_hbm.at[:, pl.ds(0, 128)], stage_v.at[k], sems.at[k]
            ).wait()

        def tr_body(g, _):
            rvec = lax.iota(jnp.int32, _LANES) + g * _LANES

            def tr_c(c, _):
                v = plsc.load_gather(res_v, [rvec, jnp.full((_LANES,), c, jnp.int32)])
                outb_v[c, pl.ds(g * _LANES, _LANES)] = v
                return ()

            lax.fori_loop(0, embed_dim, tr_c, (), unroll=4)
            return ()

        lax.fori_loop(0, b_per_w // _LANES, tr_body, (), unroll=1)
        pltpu.sync_copy(outb_v, out_hbm.at[:, pl.ds(base, b_per_w)])

    out_t = emb(x.astype(jnp.int32), table_t)
    return out_t.T


# speedup vs baseline: 1.0132x; 1.0132x over previous
import functools

import jax
import jax.numpy as jnp
from jax import lax
from jax.experimental import pallas as pl
from jax.experimental.pallas import tpu as pltpu
from jax.experimental.pallas import tpu_sc as plsc

_NUM_CORES = 2
_NUM_SUBCORES = 16
_NUM_WORKERS = _NUM_CORES * _NUM_SUBCORES
_LANES = 16
_CHUNK = 16


def kernel(x, table):
    (batch,) = x.shape
    n_rows, embed_dim = table.shape
    table_t = table.T
    b_per_w = batch // _NUM_WORKERS
    mesh = plsc.VectorSubcoreMesh(core_axis_name="c", subcore_axis_name="s")

    @functools.partial(
        pl.kernel,
        mesh=mesh,
        out_type=jax.ShapeDtypeStruct((embed_dim, batch), table.dtype),
        scratch_types=[
            pltpu.VMEM((b_per_w,), jnp.int32),
            pltpu.VMEM((_CHUNK, embed_dim, 128), jnp.float32),
            pltpu.VMEM((embed_dim, b_per_w), jnp.float32),
            pltpu.SemaphoreType.DMA,
            pltpu.SemaphoreType.DMA,
        ],
        compiler_params=pltpu.CompilerParams(needs_layout_passes=False),
    )
    def emb(x_hbm, table_hbm, out_hbm, idx_v, stage_v, outb_v, sem_in, sem_out):
        wid = lax.axis_index("s") * _NUM_CORES + lax.axis_index("c")
        base = wid * b_per_w
        pltpu.sync_copy(x_hbm.at[pl.ds(base, b_per_w)], idx_v)

        def chunk_body(ci, _):
            cbase = ci * _CHUNK

            ivec = idx_v[pl.ds(cbase, _CHUNK)]
            for k in range(_CHUNK):
                col = pl.multiple_of((ivec[k] // 128) * 128, 128)
                pltpu.async_copy(
                    table_hbm.at[:, pl.ds(col, 128)], stage_v.at[k], sem_in
                )

            def drain(k, _):
                pltpu.make_async_copy(
                    table_hbm.at[:, pl.ds(0, 128)], stage_v.at[k], sem_in
                ).wait()
                return ()

            lax.fori_loop(0, _CHUNK, drain, (), unroll=8)

            def extract(g, _):
                idx16 = idx_v[pl.ds(cbase + g * _LANES, _LANES)]
                lane = lax.rem(idx16, 128)
                kvec = lax.iota(jnp.int32, _LANES) + g * _LANES

                def comp(c, _):
                    cvec = jnp.full((_LANES,), c, dtype=jnp.int32)
                    v = plsc.load_gather(stage_v, [kvec, cvec, lane])
                    outb_v[c, pl.ds(cbase + g * _LANES, _LANES)] = v
                    return ()

                lax.fori_loop(0, embed_dim, comp, (), unroll=8)
                return ()

            lax.fori_loop(0, _CHUNK // _LANES, extract, (), unroll=1)
            return ()

        lax.fori_loop(0, b_per_w // _CHUNK, chunk_body, (), unroll=1)
        pltpu.sync_copy(outb_v, out_hbm.at[:, pl.ds(base, b_per_w)])

    out_t = emb(x.astype(jnp.int32), table_t)
    return out_t.T
